# baseline (speedup 1.0000x reference)
import functools

import jax
import jax.numpy as jnp
from jax import lax
from jax.experimental import pallas as pl
from jax.experimental.pallas import tpu as pltpu
from jax.experimental.pallas import tpu_sc as plsc

N_CONTENT = 10000
N_USER = 10000
E = 320000
D = 128

NC = 2
NS = 16
NW = NC * NS
EPT = E // NW
C = 40
NCHUNK = EPT // C
NB = 5
NSEG = 5
CPS = NCHUNK // NSEG
RPT = N_CONTENT // NS
NZF = RPT // C
ZT = RPT - NZF * C
CW = 16
CB = 125


def _bdot(a, b):
    return jnp.dot(a.astype(jnp.bfloat16), b.astype(jnp.bfloat16),
                   preferred_element_type=jnp.float32)


def _lin_body(x_ref, w_ref, b_ref, o_ref):
    h = _bdot(x_ref[...], w_ref[...])
    o_ref[...] = jnp.maximum(h + b_ref[...], 0.0)


def _lin_user(x_user, W_user, b_user):
    RA = 2000
    return pl.pallas_call(
        _lin_body,
        grid=(N_USER // RA,),
        in_specs=[
            pl.BlockSpec((RA, D), lambda i: (i, 0)),
            pl.BlockSpec((D, D), lambda i: (0, 0)),
            pl.BlockSpec((1, D), lambda i: (0, 0)),
        ],
        out_specs=pl.BlockSpec((RA, D), lambda i: (i, 0)),
        out_shape=jax.ShapeDtypeStruct((N_USER, D), jnp.float32),
    )(x_user, W_user, b_user.reshape(1, D))


def _sc_body(hu, eidx, out0, out1, cout0, cout1, sidx0, didx0, sidx1, didx1,
             rows0, rows1, rows2, rows3, rows4, ones, cbuf, asum, acnt,
             gsem0, gsem1, gsem2, gsem3, gsem4,
             ssem0, ssem1, ssem2, ssem3, ssem4, isem0, isem1):
    cid = lax.axis_index("c")
    sid = lax.axis_index("s")
    wid = sid * NC + cid
    rows = (rows0, rows1, rows2, rows3, rows4)
    gsem = (gsem0, gsem1, gsem2, gsem3, gsem4)
    ssem = (ssem0, ssem1, ssem2, ssem3, ssem4)
    sidx = (sidx0, sidx1)
    didx = (didx0, didx1)
    isem = (isem0, isem1)

    zero16 = jnp.zeros((16,), jnp.float32)
    one16 = jnp.ones((16,), jnp.float32)

    def _orow(r, carry):
        ones[r, pl.ds(0, CW)] = one16
        return carry

    lax.fori_loop(0, C, _orow, 0)

    def _zrow(r, carry):
        for j in range(D // 16):
            rows0[r, pl.ds(j * 16, 16)] = zero16
        return carry

    lax.fori_loop(0, C, _zrow, 0)

    def _crow(r, carry):
        cbuf[r, pl.ds(0, CW)] = zero16
        return carry

    lax.fori_loop(0, CB, _crow, 0)

    base = sid * RPT
    for k in range(NZF):
        pltpu.async_copy(rows0, asum.at[pl.ds(base + k * C, C)], gsem0)
    pltpu.async_copy(rows0.at[pl.ds(0, ZT)],
                     asum.at[pl.ds(base + NZF * C, ZT)], gsem0)
    for k in range(RPT // CB):
        pltpu.async_copy(cbuf, acnt.at[pl.ds(base + k * CB, CB)], gsem1)
    for k in range(NZF):
        pltpu.make_async_copy(rows0, asum.at[pl.ds(base, C)], gsem0).wait()
    pltpu.make_async_copy(rows0.at[pl.ds(0, ZT)],
                          asum.at[pl.ds(base, ZT)], gsem0).wait()
    for k in range(RPT // CB):
        pltpu.make_async_copy(cbuf, acnt.at[pl.ds(base, CB)], gsem1).wait()
    plsc.subcore_barrier()

    def _stage_idx(s, sync):
        st = s % 2
        blk = pl.ds(s * CPS, CPS)
        if sync:
            pltpu.sync_copy(eidx.at[0, wid, blk], sidx[st])
            pltpu.sync_copy(eidx.at[1, wid, blk], didx[st])
        else:
            pltpu.async_copy(eidx.at[0, wid, blk], sidx[st], isem[st])
            pltpu.async_copy(eidx.at[1, wid, blk], didx[st], isem[st])

    def _wait_idx(s):
        st = s % 2
        pltpu.make_async_copy(eidx.at[0, wid, pl.ds(0, CPS)], sidx[st],
                              isem[st]).wait()
        pltpu.make_async_copy(eidx.at[1, wid, pl.ds(0, CPS)], didx[st],
                              isem[st]).wait()

    def _issue_gather(st, ci, b):
        pltpu.async_copy(hu.at[sidx[st].at[ci]], rows[b], gsem[b])

    def _wait_gather(b):
        pltpu.make_async_copy(hu.at[sidx[0].at[0]], rows[b], gsem[b]).wait()

    def _issue_scatter(st, ci, b):
        pltpu.async_copy(rows[b], asum.at[didx[st].at[ci]], ssem[b], add=True)
        pltpu.async_copy(ones, acnt.at[didx[st].at[ci]], ssem[b], add=True)

    def _wait_scatter(b):
        pltpu.make_async_copy(rows[b], asum.at[didx[0].at[0]],
                              ssem[b]).wait()
        pltpu.make_async_copy(ones, acnt.at[didx[0].at[0]], ssem[b]).wait()

    _stage_idx(0, sync=True)
    for b in range(NB):
        _issue_gather(0, b, b)
    for s in range(NSEG):
        st = s % 2
        if s + 1 < NSEG:
            _stage_idx(s + 1, sync=False)

        def _ring(pi, carry):
            for b in range(NB):
                c = pi * NB + b
                _wait_gather(b)
                _issue_scatter(st, c, b)
                _wait_scatter(b)
                _issue_gather(st, c + NB, b)
            return carry

        lax.fori_loop(0, CPS // NB - 1, _ring, 0)
        if s + 1 < NSEG:
            _wait_idx(s + 1)
        for b in range(NB):
            _wait_gather(b)
            _issue_scatter(st, CPS - NB + b, b)
            _wait_scatter(b)
            if s + 1 < NSEG:
                _issue_gather((s + 1) % 2, b, b)
    plsc.subcore_barrier()

    for k in range(RPT // CB):
        r0 = base + k * CB
        pltpu.sync_copy(acnt.at[pl.ds(r0, CB)], cbuf)

        @pl.when(cid == 0)
        def _():
            pltpu.async_copy(cbuf, cout0.at[pl.ds(r0, CB)], ssem0)

        @pl.when(cid == 1)
        def _():
            pltpu.async_copy(cbuf, cout1.at[pl.ds(r0, CB)], ssem0)

        pltpu.make_async_copy(cbuf, cout0.at[pl.ds(r0, CB)], ssem0).wait()
    for k in range(NZF + 1):
        b = k % NB
        n = C if k < NZF else ZT
        r0 = base + k * C
        if k >= NB:
            pltpu.make_async_copy(rows[b], out0.at[pl.ds(0, C)],
                                  gsem[b]).wait()
        bounce = rows[b] if n == C else rows[b].at[pl.ds(0, ZT)]
        pltpu.sync_copy(asum.at[pl.ds(r0, n)], bounce)

        @pl.when(cid == 0)
        def _():
            pltpu.async_copy(bounce, out0.at[pl.ds(r0, n)], gsem[b])

        @pl.when(cid == 1)
        def _():
            pltpu.async_copy(bounce, out1.at[pl.ds(r0, n)], gsem[b])

    for k in range(NB):
        n = C if (NZF + 1 - NB + k) < NZF else ZT
        pltpu.make_async_copy(rows[0].at[pl.ds(0, n)],
                              out0.at[pl.ds(0, n)],
                              gsem[(NZF + 1 - NB + k) % NB]).wait()


_sc_seg_sum = functools.partial(
    pl.kernel,
    out_type=(jax.ShapeDtypeStruct((N_CONTENT, D), jnp.float32),
              jax.ShapeDtypeStruct((N_CONTENT, D), jnp.float32),
              jax.ShapeDtypeStruct((N_CONTENT, CW), jnp.float32),
              jax.ShapeDtypeStruct((N_CONTENT, CW), jnp.float32)),
    mesh=plsc.VectorSubcoreMesh(core_axis_name="c", subcore_axis_name="s"),
    scratch_types=[
        pltpu.VMEM((CPS, C), jnp.int32),
        pltpu.VMEM((CPS, C), jnp.int32),
        pltpu.VMEM((CPS, C), jnp.int32),
        pltpu.VMEM((CPS, C), jnp.int32),
    ] + [pltpu.VMEM((C, D), jnp.float32) for _ in range(NB)] + [
        pltpu.VMEM((C, CW), jnp.float32),
        pltpu.VMEM((CB, CW), jnp.float32),
        pltpu.VMEM_SHARED((N_CONTENT, D), jnp.float32),
        pltpu.VMEM_SHARED((N_CONTENT, CW), jnp.float32),
    ] + [pltpu.SemaphoreType.DMA for _ in range(2 * NB + 2)],
    compiler_params=pltpu.CompilerParams(use_tc_tiling_on_sc=False),
)(_sc_body)


_RB = 2000


def _final_body(p0_ref, p1_ref, c0_hbm, c1_hbm, xc_ref, wc_ref, bc_ref,
                wl_ref, bl_ref, wr_ref, wo_ref, bo_ref, o_ref,
                c0_v, c1_v, csem):
    i = pl.program_id(0)
    cp0 = pltpu.make_async_copy(c0_hbm.at[pl.ds(i * _RB, _RB)], c0_v, csem)
    cp0.start()
    cp1 = pltpu.make_async_copy(c1_hbm.at[pl.ds(i * _RB, _RB)], c1_v, csem)
    cp1.start()
    hc = jnp.maximum(_bdot(xc_ref[...], wc_ref[...]) + bc_ref[...], 0.0)
    wo = wo_ref[...]
    wlo = _bdot(wl_ref[...], wo)
    wro = _bdot(wr_ref[...], wo)
    bb = _bdot(bl_ref[...], wo) + bo_ref[...]
    s = p0_ref[...] + p1_ref[...]
    cp0.wait()
    cp1.wait()
    cnt = c0_v[:, 0:1] + c1_v[:, 0:1]
    mean = s / jnp.maximum(cnt, 1.0)
    o_ref[...] = _bdot(mean, wlo) + _bdot(hc, wro) + bb


def _final(p0, p1, c0, c1, x_content, W_content, b_content, W_l, b_l, W_r,
           W_out, b_out):
    full = lambda i: (0, 0)
    return pl.pallas_call(
        _final_body,
        grid=(N_CONTENT // _RB,),
        in_specs=[
            pl.BlockSpec((_RB, D), lambda i: (i, 0)),
            pl.BlockSpec((_RB, D), lambda i: (i, 0)),
            pl.BlockSpec(memory_space=pl.ANY),
            pl.BlockSpec(memory_space=pl.ANY),
            pl.BlockSpec((_RB, D), lambda i: (i, 0)),
            pl.BlockSpec((D, D), full),
            pl.BlockSpec((1, D), full),
            pl.BlockSpec((D, D), full),
            pl.BlockSpec((1, D), full),
            pl.BlockSpec((D, D), full),
            pl.BlockSpec((D, D), full),
            pl.BlockSpec((1, D), full),
        ],
        out_specs=pl.BlockSpec((_RB, D), lambda i: (i, 0)),
        out_shape=jax.ShapeDtypeStruct((N_CONTENT, D), jnp.float32),
        scratch_shapes=[
            pltpu.VMEM((_RB, CW), jnp.float32),
            pltpu.VMEM((_RB, CW), jnp.float32),
            pltpu.SemaphoreType.DMA,
        ],
    )(p0, p1, c0, c1, x_content, W_content, b_content.reshape(1, D),
      W_l, b_l.reshape(1, D), W_r, W_out, b_out.reshape(1, D))


def kernel(x_content, x_user, edge_index, W_content, b_content, W_user, b_user,
           W_l, b_l, W_r, W_out, b_out):
    hu = _lin_user(x_user, W_user, b_user)
    eidx = edge_index.reshape(2, NW, NCHUNK, C)
    p0, p1, c0, c1 = _sc_seg_sum(hu, eidx)
    return _final(p0, p1, c0, c1, x_content, W_content, b_content, W_l, b_l,
                  W_r, W_out, b_out)


# speedup vs baseline: 1.0386x; 1.0386x over previous
import functools

import jax
import jax.numpy as jnp
from jax import lax
from jax.experimental import pallas as pl
from jax.experimental.pallas import tpu as pltpu
from jax.experimental.pallas import tpu_sc as plsc

N_CONTENT = 10000
N_USER = 10000
E = 320000
D = 128

NC = 2
NS = 16
NW = NC * NS
EPT = E // NW
C = 40
NCHUNK = EPT // C
NB = 5
NSEG = 5
CPS = NCHUNK // NSEG
RPT = N_CONTENT // NS
NZF = RPT // C
ZT = RPT - NZF * C
CW = 16
CB = 125


def _bdot(a, b):
    return jnp.dot(a.astype(jnp.bfloat16), b.astype(jnp.bfloat16),
                   preferred_element_type=jnp.float32)


def _lin_body(x_ref, w_ref, b_ref, o_ref):
    h = _bdot(x_ref[...], w_ref[...])
    o_ref[...] = jnp.maximum(h + b_ref[...], 0.0)


def _lin_user(x_user, W_user, b_user):
    RA = 2000
    return pl.pallas_call(
        _lin_body,
        grid=(N_USER // RA,),
        in_specs=[
            pl.BlockSpec((RA, D), lambda i: (i, 0)),
            pl.BlockSpec((D, D), lambda i: (0, 0)),
            pl.BlockSpec((1, D), lambda i: (0, 0)),
        ],
        out_specs=pl.BlockSpec((RA, D), lambda i: (i, 0)),
        out_shape=jax.ShapeDtypeStruct((N_USER, D), jnp.float32),
    )(x_user, W_user, b_user.reshape(1, D))


def _sc_body(hu, eidx, out0, out1, cout0, cout1, sidx0, didx0, sidx1, didx1,
             rows0, rows1, rows2, rows3, rows4, ones, cbuf, asum, acnt,
             gsem0, gsem1, gsem2, gsem3, gsem4,
             ssem0, ssem1, ssem2, ssem3, ssem4, isem0, isem1):
    cid = lax.axis_index("c")
    sid = lax.axis_index("s")
    wid = sid * NC + cid
    rows = (rows0, rows1, rows2, rows3, rows4)
    gsem = (gsem0, gsem1, gsem2, gsem3, gsem4)
    ssem = (ssem0, ssem1, ssem2, ssem3, ssem4)
    sidx = (sidx0, sidx1)
    didx = (didx0, didx1)
    isem = (isem0, isem1)

    zero16 = jnp.zeros((16,), jnp.float32)
    one16 = jnp.ones((16,), jnp.float32)

    def _orow(r, carry):
        ones[r, pl.ds(0, CW)] = one16
        return carry

    lax.fori_loop(0, C, _orow, 0)

    def _zrow(r, carry):
        for j in range(D // 16):
            rows0[r, pl.ds(j * 16, 16)] = zero16
        return carry

    lax.fori_loop(0, C, _zrow, 0)

    def _crow(r, carry):
        cbuf[r, pl.ds(0, CW)] = zero16
        return carry

    lax.fori_loop(0, CB, _crow, 0)

    base = sid * RPT
    for k in range(NZF):
        pltpu.async_copy(rows0, asum.at[pl.ds(base + k * C, C)], gsem0)
    pltpu.async_copy(rows0.at[pl.ds(0, ZT)],
                     asum.at[pl.ds(base + NZF * C, ZT)], gsem0)
    for k in range(RPT // CB):
        pltpu.async_copy(cbuf, acnt.at[pl.ds(base + k * CB, CB)], gsem1)
    for k in range(NZF):
        pltpu.make_async_copy(rows0, asum.at[pl.ds(base, C)], gsem0).wait()
    pltpu.make_async_copy(rows0.at[pl.ds(0, ZT)],
                          asum.at[pl.ds(base, ZT)], gsem0).wait()
    for k in range(RPT // CB):
        pltpu.make_async_copy(cbuf, acnt.at[pl.ds(base, CB)], gsem1).wait()
    plsc.subcore_barrier()

    def _stage_idx(s, sync):
        st = s % 2
        blk = pl.ds(s * CPS, CPS)
        if sync:
            pltpu.sync_copy(eidx.at[0, wid, blk], sidx[st])
            pltpu.sync_copy(eidx.at[1, wid, blk], didx[st])
        else:
            pltpu.async_copy(eidx.at[0, wid, blk], sidx[st], isem[st])
            pltpu.async_copy(eidx.at[1, wid, blk], didx[st], isem[st])

    def _wait_idx(s):
        st = s % 2
        pltpu.make_async_copy(eidx.at[0, wid, pl.ds(0, CPS)], sidx[st],
                              isem[st]).wait()
        pltpu.make_async_copy(eidx.at[1, wid, pl.ds(0, CPS)], didx[st],
                              isem[st]).wait()

    def _issue_gather(st, ci, b):
        pltpu.async_copy(hu.at[sidx[st].at[ci]], rows[b], gsem[b])

    def _wait_gather(b):
        pltpu.make_async_copy(hu.at[sidx[0].at[0]], rows[b], gsem[b]).wait()

    def _issue_scatter(st, ci, b):
        pltpu.async_copy(rows[b], asum.at[didx[st].at[ci]], ssem[b], add=True)
        pltpu.async_copy(ones, acnt.at[didx[st].at[ci]], ssem[b], add=True)

    def _wait_scatter(b):
        pltpu.make_async_copy(rows[b], asum.at[didx[0].at[0]],
                              ssem[b]).wait()
        pltpu.make_async_copy(ones, acnt.at[didx[0].at[0]], ssem[b]).wait()

    _stage_idx(0, sync=True)
    for b in range(NB):
        _issue_gather(0, b, b)
    for s in range(NSEG):
        st = s % 2
        if s + 1 < NSEG:
            _stage_idx(s + 1, sync=False)

        def _ring(pi, carry):
            for b in range(NB):
                c = pi * NB + b
                _wait_gather(b)
                _issue_scatter(st, c, b)
                _wait_scatter(b)
                _issue_gather(st, c + NB, b)
            return carry

        lax.fori_loop(0, CPS // NB - 1, _ring, 0)
        if s + 1 < NSEG:
            _wait_idx(s + 1)
        for b in range(NB):
            _wait_gather(b)
            _issue_scatter(st, CPS - NB + b, b)
            _wait_scatter(b)
            if s + 1 < NSEG:
                _issue_gather((s + 1) % 2, b, b)
    plsc.subcore_barrier()

    for k in range(RPT // CB):
        r0 = base + k * CB
        pltpu.sync_copy(acnt.at[pl.ds(r0, CB)], cbuf)

        @pl.when(cid == 0)
        def _():
            pltpu.async_copy(cbuf, cout0.at[pl.ds(r0, CB)], ssem0)

        @pl.when(cid == 1)
        def _():
            pltpu.async_copy(cbuf, cout1.at[pl.ds(r0, CB)], ssem0)

        pltpu.make_async_copy(cbuf, cout0.at[pl.ds(r0, CB)], ssem0).wait()
    for k in range(NZF + 1):
        b = k % NB
        n = C if k < NZF else ZT
        r0 = base + k * C
        if k >= NB:
            pltpu.make_async_copy(rows[b], out0.at[pl.ds(0, C)],
                                  gsem[b]).wait()
        bounce = rows[b] if n == C else rows[b].at[pl.ds(0, ZT)]
        pltpu.sync_copy(asum.at[pl.ds(r0, n)], bounce)

        @pl.when(cid == 0)
        def _():
            pltpu.async_copy(bounce, out0.at[pl.ds(r0, n)], gsem[b])

        @pl.when(cid == 1)
        def _():
            pltpu.async_copy(bounce, out1.at[pl.ds(r0, n)], gsem[b])

    for k in range(NB):
        n = C if (NZF + 1 - NB + k) < NZF else ZT
        pltpu.make_async_copy(rows[0].at[pl.ds(0, n)],
                              out0.at[pl.ds(0, n)],
                              gsem[(NZF + 1 - NB + k) % NB]).wait()


_sc_seg_sum = functools.partial(
    pl.kernel,
    out_type=(jax.ShapeDtypeStruct((N_CONTENT, D), jnp.float32),
              jax.ShapeDtypeStruct((N_CONTENT, D), jnp.float32),
              jax.ShapeDtypeStruct((N_CONTENT, CW), jnp.float32),
              jax.ShapeDtypeStruct((N_CONTENT, CW), jnp.float32)),
    mesh=plsc.VectorSubcoreMesh(core_axis_name="c", subcore_axis_name="s"),
    scratch_types=[
        pltpu.VMEM((CPS, C), jnp.int32),
        pltpu.VMEM((CPS, C), jnp.int32),
        pltpu.VMEM((CPS, C), jnp.int32),
        pltpu.VMEM((CPS, C), jnp.int32),
    ] + [pltpu.VMEM((C, D), jnp.float32) for _ in range(NB)] + [
        pltpu.VMEM((C, CW), jnp.float32),
        pltpu.VMEM((CB, CW), jnp.float32),
        pltpu.VMEM_SHARED((N_CONTENT, D), jnp.float32),
        pltpu.VMEM_SHARED((N_CONTENT, CW), jnp.float32),
    ] + [pltpu.SemaphoreType.DMA for _ in range(2 * NB + 2)],
    compiler_params=pltpu.CompilerParams(use_tc_tiling_on_sc=False),
)(_sc_body)


_RB = 2000


def _final_body(p0_ref, p1_ref, c0_ref, c1_ref, xc_ref, wc_ref, bc_ref,
                wl_ref, bl_ref, wr_ref, wo_ref, bo_ref, o_ref):
    hc = jnp.maximum(_bdot(xc_ref[...], wc_ref[...]) + bc_ref[...], 0.0)
    wo = wo_ref[...]
    wlo = _bdot(wl_ref[...], wo)
    wro = _bdot(wr_ref[...], wo)
    bb = _bdot(bl_ref[...], wo) + bo_ref[...]
    s = p0_ref[...] + p1_ref[...]
    cnt = c0_ref[:, 0:1] + c1_ref[:, 0:1]
    mean = s / jnp.maximum(cnt, 1.0)
    o_ref[...] = _bdot(mean, wlo) + _bdot(hc, wro) + bb


def _final(p0, p1, c0, c1, x_content, W_content, b_content, W_l, b_l, W_r,
           W_out, b_out):
    full = lambda i: (0, 0)
    return pl.pallas_call(
        _final_body,
        grid=(N_CONTENT // _RB,),
        in_specs=[
            pl.BlockSpec((_RB, D), lambda i: (i, 0)),
            pl.BlockSpec((_RB, D), lambda i: (i, 0)),
            pl.BlockSpec((_RB, CW), lambda i: (i, 0)),
            pl.BlockSpec((_RB, CW), lambda i: (i, 0)),
            pl.BlockSpec((_RB, D), lambda i: (i, 0)),
            pl.BlockSpec((D, D), full),
            pl.BlockSpec((1, D), full),
            pl.BlockSpec((D, D), full),
            pl.BlockSpec((1, D), full),
            pl.BlockSpec((D, D), full),
            pl.BlockSpec((D, D), full),
            pl.BlockSpec((1, D), full),
        ],
        out_specs=pl.BlockSpec((_RB, D), lambda i: (i, 0)),
        out_shape=jax.ShapeDtypeStruct((N_CONTENT, D), jnp.float32),
    )(p0, p1, c0, c1, x_content, W_content, b_content.reshape(1, D),
      W_l, b_l.reshape(1, D), W_r, W_out, b_out.reshape(1, D))


def kernel(x_content, x_user, edge_index, W_content, b_content, W_user, b_user,
           W_l, b_l, W_r, W_out, b_out):
    hu = _lin_user(x_user, W_user, b_user)
    eidx = edge_index.reshape(2, NW, NCHUNK, C)
    p0, p1, c0, c1 = _sc_seg_sum(hu, eidx)
    return _final(p0, p1, c0, c1, x_content, W_content, b_content, W_l, b_l,
                  W_r, W_out, b_out)


# speedup vs baseline: 1.1366x; 1.0944x over previous
import functools

import jax
import jax.numpy as jnp
from jax import lax
from jax.experimental import pallas as pl
from jax.experimental.pallas import tpu as pltpu
from jax.experimental.pallas import tpu_sc as plsc

N_CONTENT = 10000
N_USER = 10000
E = 320000
D = 128

NC = 2
NS = 16
NW = NC * NS
EPT = E // NW
C = 40
NCHUNK = EPT // C
NB = 5
NSEG = 5
CPS = NCHUNK // NSEG
RPT = N_CONTENT // NS
NZF = RPT // C
ZT = RPT - NZF * C
CW = 16
CB = 125


def _bdot(a, b):
    return jnp.dot(a.astype(jnp.bfloat16), b.astype(jnp.bfloat16),
                   preferred_element_type=jnp.float32)


def _lin_body(x_ref, w_ref, b_ref, o_ref):
    h = _bdot(x_ref[...], w_ref[...])
    o_ref[...] = jnp.maximum(h + b_ref[...], 0.0).astype(o_ref.dtype)


def _lin_user(x_user, W_user, b_user):
    RA = 2000
    return pl.pallas_call(
        _lin_body,
        grid=(N_USER // RA,),
        in_specs=[
            pl.BlockSpec((RA, D), lambda i: (i, 0)),
            pl.BlockSpec((D, D), lambda i: (0, 0)),
            pl.BlockSpec((1, D), lambda i: (0, 0)),
        ],
        out_specs=pl.BlockSpec((RA, D), lambda i: (i, 0)),
        out_shape=jax.ShapeDtypeStruct((N_USER, D), jnp.bfloat16),
    )(x_user, W_user, b_user.reshape(1, D))


def _sc_body(hu, eidx, out0, out1, cout0, cout1, sidx0, didx0, sidx1, didx1,
             rows0, rows1, rows2, rows3, rows4, ones, cbuf, asum, acnt,
             gsem0, gsem1, gsem2, gsem3, gsem4,
             ssem0, ssem1, ssem2, ssem3, ssem4, isem0, isem1):
    cid = lax.axis_index("c")
    sid = lax.axis_index("s")
    wid = sid * NC + cid
    rows = (rows0, rows1, rows2, rows3, rows4)
    gsem = (gsem0, gsem1, gsem2, gsem3, gsem4)
    ssem = (ssem0, ssem1, ssem2, ssem3, ssem4)
    sidx = (sidx0, sidx1)
    didx = (didx0, didx1)
    isem = (isem0, isem1)

    zero16 = jnp.zeros((16,), jnp.float32)
    one16 = jnp.ones((16,), jnp.float32)

    def _orow(r, carry):
        ones[r, pl.ds(0, CW)] = one16
        return carry

    lax.fori_loop(0, C, _orow, 0)

    zero32b = jnp.zeros((32,), jnp.bfloat16)

    def _zrow(r, carry):
        for j in range(D // 32):
            rows0[r, pl.ds(j * 32, 32)] = zero32b
        return carry

    lax.fori_loop(0, C, _zrow, 0)

    def _crow(r, carry):
        cbuf[r, pl.ds(0, CW)] = zero16
        return carry

    lax.fori_loop(0, CB, _crow, 0)

    base = sid * RPT
    for k in range(NZF):
        pltpu.async_copy(rows0, asum.at[pl.ds(base + k * C, C)], gsem0)
    pltpu.async_copy(rows0.at[pl.ds(0, ZT)],
                     asum.at[pl.ds(base + NZF * C, ZT)], gsem0)
    for k in range(RPT // CB):
        pltpu.async_copy(cbuf, acnt.at[pl.ds(base + k * CB, CB)], gsem1)
    for k in range(NZF):
        pltpu.make_async_copy(rows0, asum.at[pl.ds(base, C)], gsem0).wait()
    pltpu.make_async_copy(rows0.at[pl.ds(0, ZT)],
                          asum.at[pl.ds(base, ZT)], gsem0).wait()
    for k in range(RPT // CB):
        pltpu.make_async_copy(cbuf, acnt.at[pl.ds(base, CB)], gsem1).wait()
    plsc.subcore_barrier()

    def _stage_idx(s, sync):
        st = s % 2
        blk = pl.ds(s * CPS, CPS)
        if sync:
            pltpu.sync_copy(eidx.at[0, wid, blk], sidx[st])
            pltpu.sync_copy(eidx.at[1, wid, blk], didx[st])
        else:
            pltpu.async_copy(eidx.at[0, wid, blk], sidx[st], isem[st])
            pltpu.async_copy(eidx.at[1, wid, blk], didx[st], isem[st])

    def _wait_idx(s):
        st = s % 2
        pltpu.make_async_copy(eidx.at[0, wid, pl.ds(0, CPS)], sidx[st],
                              isem[st]).wait()
        pltpu.make_async_copy(eidx.at[1, wid, pl.ds(0, CPS)], didx[st],
                              isem[st]).wait()

    def _issue_gather(st, ci, b):
        pltpu.async_copy(hu.at[sidx[st].at[ci]], rows[b], gsem[b])

    def _wait_gather(b):
        pltpu.make_async_copy(hu.at[sidx[0].at[0]], rows[b], gsem[b]).wait()

    def _issue_scatter(st, ci, b):
        pltpu.async_copy(rows[b], asum.at[didx[st].at[ci]], ssem[b], add=True)
        pltpu.async_copy(ones, acnt.at[didx[st].at[ci]], ssem[b], add=True)

    def _wait_scatter(b):
        pltpu.make_async_copy(rows[b], asum.at[didx[0].at[0]],
                              ssem[b]).wait()
        pltpu.make_async_copy(ones, acnt.at[didx[0].at[0]], ssem[b]).wait()

    _stage_idx(0, sync=True)
    for b in range(NB):
        _issue_gather(0, b, b)
    for s in range(NSEG):
        st = s % 2
        if s + 1 < NSEG:
            _stage_idx(s + 1, sync=False)

        def _ring(pi, carry):
            for b in range(NB):
                c = pi * NB + b
                _wait_gather(b)
                _issue_scatter(st, c, b)
                _wait_scatter(b)
                _issue_gather(st, c + NB, b)
            return carry

        lax.fori_loop(0, CPS // NB - 1, _ring, 0)
        if s + 1 < NSEG:
            _wait_idx(s + 1)
        for b in range(NB):
            _wait_gather(b)
            _issue_scatter(st, CPS - NB + b, b)
            _wait_scatter(b)
            if s + 1 < NSEG:
                _issue_gather((s + 1) % 2, b, b)
    plsc.subcore_barrier()

    for k in range(RPT // CB):
        r0 = base + k * CB
        pltpu.sync_copy(acnt.at[pl.ds(r0, CB)], cbuf)

        @pl.when(cid == 0)
        def _():
            pltpu.async_copy(cbuf, cout0.at[pl.ds(r0, CB)], ssem0)

        @pl.when(cid == 1)
        def _():
            pltpu.async_copy(cbuf, cout1.at[pl.ds(r0, CB)], ssem0)

        pltpu.make_async_copy(cbuf, cout0.at[pl.ds(r0, CB)], ssem0).wait()
    for k in range(NZF + 1):
        b = k % NB
        n = C if k < NZF else ZT
        r0 = base + k * C
        if k >= NB:
            pltpu.make_async_copy(rows[b], out0.at[pl.ds(0, C)],
                                  gsem[b]).wait()
        bounce = rows[b] if n == C else rows[b].at[pl.ds(0, ZT)]
        pltpu.sync_copy(asum.at[pl.ds(r0, n)], bounce)

        @pl.when(cid == 0)
        def _():
            pltpu.async_copy(bounce, out0.at[pl.ds(r0, n)], gsem[b])

        @pl.when(cid == 1)
        def _():
            pltpu.async_copy(bounce, out1.at[pl.ds(r0, n)], gsem[b])

    for k in range(NB):
        n = C if (NZF + 1 - NB + k) < NZF else ZT
        pltpu.make_async_copy(rows[0].at[pl.ds(0, n)],
                              out0.at[pl.ds(0, n)],
                              gsem[(NZF + 1 - NB + k) % NB]).wait()


_sc_seg_sum = functools.partial(
    pl.kernel,
    out_type=(jax.ShapeDtypeStruct((N_CONTENT, D), jnp.bfloat16),
              jax.ShapeDtypeStruct((N_CONTENT, D), jnp.bfloat16),
              jax.ShapeDtypeStruct((N_CONTENT, CW), jnp.float32),
              jax.ShapeDtypeStruct((N_CONTENT, CW), jnp.float32)),
    mesh=plsc.VectorSubcoreMesh(core_axis_name="c", subcore_axis_name="s"),
    scratch_types=[
        pltpu.VMEM((CPS, C), jnp.int32),
        pltpu.VMEM((CPS, C), jnp.int32),
        pltpu.VMEM((CPS, C), jnp.int32),
        pltpu.VMEM((CPS, C), jnp.int32),
    ] + [pltpu.VMEM((C, D), jnp.bfloat16) for _ in range(NB)] + [
        pltpu.VMEM((C, CW), jnp.float32),
        pltpu.VMEM((CB, CW), jnp.float32),
        pltpu.VMEM_SHARED((N_CONTENT, D), jnp.bfloat16),
        pltpu.VMEM_SHARED((N_CONTENT, CW), jnp.float32),
    ] + [pltpu.SemaphoreType.DMA for _ in range(2 * NB + 2)],
    compiler_params=pltpu.CompilerParams(use_tc_tiling_on_sc=False),
)(_sc_body)


_RB = 2000


def _final_body(p0_ref, p1_ref, c0_ref, c1_ref, xc_ref, wc_ref, bc_ref,
                wl_ref, bl_ref, wr_ref, wo_ref, bo_ref, o_ref):
    hc = jnp.maximum(_bdot(xc_ref[...], wc_ref[...]) + bc_ref[...], 0.0)
    wo = wo_ref[...]
    wlo = _bdot(wl_ref[...], wo)
    wro = _bdot(wr_ref[...], wo)
    bb = _bdot(bl_ref[...], wo) + bo_ref[...]
    s = (p0_ref[...].astype(jnp.float32) + p1_ref[...].astype(jnp.float32))
    cnt = c0_ref[:, 0:1] + c1_ref[:, 0:1]
    mean = s / jnp.maximum(cnt, 1.0)
    o_ref[...] = _bdot(mean, wlo) + _bdot(hc, wro) + bb


def _final(p0, p1, c0, c1, x_content, W_content, b_content, W_l, b_l, W_r,
           W_out, b_out):
    full = lambda i: (0, 0)
    return pl.pallas_call(
        _final_body,
        grid=(N_CONTENT // _RB,),
        in_specs=[
            pl.BlockSpec((_RB, D), lambda i: (i, 0)),
            pl.BlockSpec((_RB, D), lambda i: (i, 0)),
            pl.BlockSpec((_RB, CW), lambda i: (i, 0)),
            pl.BlockSpec((_RB, CW), lambda i: (i, 0)),
            pl.BlockSpec((_RB, D), lambda i: (i, 0)),
            pl.BlockSpec((D, D), full),
            pl.BlockSpec((1, D), full),
            pl.BlockSpec((D, D), full),
            pl.BlockSpec((1, D), full),
            pl.BlockSpec((D, D), full),
            pl.BlockSpec((D, D), full),
            pl.BlockSpec((1, D), full),
        ],
        out_specs=pl.BlockSpec((_RB, D), lambda i: (i, 0)),
        out_shape=jax.ShapeDtypeStruct((N_CONTENT, D), jnp.float32),
    )(p0, p1, c0, c1, x_content, W_content, b_content.reshape(1, D),
      W_l, b_l.reshape(1, D), W_r, W_out, b_out.reshape(1, D))


def kernel(x_content, x_user, edge_index, W_content, b_content, W_user, b_user,
           W_l, b_l, W_r, W_out, b_out):
    hu = _lin_user(x_user, W_user, b_user)
    eidx = edge_index.reshape(2, NW, NCHUNK, C)
    p0, p1, c0, c1 = _sc_seg_sum(hu, eidx)
    return _final(p0, p1, c0, c1, x_content, W_content, b_content, W_l, b_l,
                  W_r, W_out, b_out)


# speedup vs baseline: 1.1870x; 1.0443x over previous
import functools

import jax
import jax.numpy as jnp
from jax import lax
from jax.experimental import pallas as pl
from jax.experimental.pallas import tpu as pltpu
from jax.experimental.pallas import tpu_sc as plsc

N_CONTENT = 10000
N_USER = 10000
E = 320000
D = 128

NC = 2
NS = 16
NW = NC * NS
EPT = E // NW
C = 80
NCHUNK = EPT // C
NB = 5
NSEG = 5
CPS = NCHUNK // NSEG
RPT = N_CONTENT // NS
NZF = RPT // C
ZT = RPT - NZF * C
CW = 16
CB = 125


def _bdot(a, b):
    return jnp.dot(a.astype(jnp.bfloat16), b.astype(jnp.bfloat16),
                   preferred_element_type=jnp.float32)


def _lin_body(x_ref, w_ref, b_ref, o_ref):
    h = _bdot(x_ref[...], w_ref[...])
    o_ref[...] = jnp.maximum(h + b_ref[...], 0.0).astype(o_ref.dtype)


def _lin_user(x_user, W_user, b_user):
    RA = 2000
    return pl.pallas_call(
        _lin_body,
        grid=(N_USER // RA,),
        in_specs=[
            pl.BlockSpec((RA, D), lambda i: (i, 0)),
            pl.BlockSpec((D, D), lambda i: (0, 0)),
            pl.BlockSpec((1, D), lambda i: (0, 0)),
        ],
        out_specs=pl.BlockSpec((RA, D), lambda i: (i, 0)),
        out_shape=jax.ShapeDtypeStruct((N_USER, D), jnp.bfloat16),
    )(x_user, W_user, b_user.reshape(1, D))


def _sc_body(hu, eidx, out0, out1, cout0, cout1, sidx0, didx0, sidx1, didx1,
             rows0, rows1, rows2, rows3, rows4, ones, cbuf, asum, acnt,
             gsem0, gsem1, gsem2, gsem3, gsem4,
             ssem0, ssem1, ssem2, ssem3, ssem4, isem0, isem1):
    cid = lax.axis_index("c")
    sid = lax.axis_index("s")
    wid = sid * NC + cid
    rows = (rows0, rows1, rows2, rows3, rows4)
    gsem = (gsem0, gsem1, gsem2, gsem3, gsem4)
    ssem = (ssem0, ssem1, ssem2, ssem3, ssem4)
    sidx = (sidx0, sidx1)
    didx = (didx0, didx1)
    isem = (isem0, isem1)

    zero16 = jnp.zeros((16,), jnp.float32)
    one16 = jnp.ones((16,), jnp.float32)

    def _orow(r, carry):
        ones[r, pl.ds(0, CW)] = one16
        return carry

    lax.fori_loop(0, C, _orow, 0)

    zero32b = jnp.zeros((32,), jnp.bfloat16)

    def _zrow(r, carry):
        for j in range(D // 32):
            rows0[r, pl.ds(j * 32, 32)] = zero32b
        return carry

    lax.fori_loop(0, C, _zrow, 0)

    def _crow(r, carry):
        cbuf[r, pl.ds(0, CW)] = zero16
        return carry

    lax.fori_loop(0, CB, _crow, 0)

    base = sid * RPT
    for k in range(NZF):
        pltpu.async_copy(rows0, asum.at[pl.ds(base + k * C, C)], gsem0)
    pltpu.async_copy(rows0.at[pl.ds(0, ZT)],
                     asum.at[pl.ds(base + NZF * C, ZT)], gsem0)
    for k in range(RPT // CB):
        pltpu.async_copy(cbuf, acnt.at[pl.ds(base + k * CB, CB)], gsem1)
    for k in range(NZF):
        pltpu.make_async_copy(rows0, asum.at[pl.ds(base, C)], gsem0).wait()
    pltpu.make_async_copy(rows0.at[pl.ds(0, ZT)],
                          asum.at[pl.ds(base, ZT)], gsem0).wait()
    for k in range(RPT // CB):
        pltpu.make_async_copy(cbuf, acnt.at[pl.ds(base, CB)], gsem1).wait()
    plsc.subcore_barrier()

    def _stage_idx(s, sync):
        st = s % 2
        blk = pl.ds(s * CPS, CPS)
        if sync:
            pltpu.sync_copy(eidx.at[0, wid, blk], sidx[st])
            pltpu.sync_copy(eidx.at[1, wid, blk], didx[st])
        else:
            pltpu.async_copy(eidx.at[0, wid, blk], sidx[st], isem[st])
            pltpu.async_copy(eidx.at[1, wid, blk], didx[st], isem[st])

    def _wait_idx(s):
        st = s % 2
        pltpu.make_async_copy(eidx.at[0, wid, pl.ds(0, CPS)], sidx[st],
                              isem[st]).wait()
        pltpu.make_async_copy(eidx.at[1, wid, pl.ds(0, CPS)], didx[st],
                              isem[st]).wait()

    def _issue_gather(st, ci, b):
        pltpu.async_copy(hu.at[sidx[st].at[ci]], rows[b], gsem[b])

    def _wait_gather(b):
        pltpu.make_async_copy(hu.at[sidx[0].at[0]], rows[b], gsem[b]).wait()

    def _issue_scatter(st, ci, b):
        pltpu.async_copy(rows[b], asum.at[didx[st].at[ci]], ssem[b], add=True)
        pltpu.async_copy(ones, acnt.at[didx[st].at[ci]], ssem[b], add=True)

    def _wait_scatter(b):
        pltpu.make_async_copy(rows[b], asum.at[didx[0].at[0]],
                              ssem[b]).wait()
        pltpu.make_async_copy(ones, acnt.at[didx[0].at[0]], ssem[b]).wait()

    _stage_idx(0, sync=True)
    for b in range(NB):
        _issue_gather(0, b, b)
    for s in range(NSEG):
        st = s % 2
        if s + 1 < NSEG:
            _stage_idx(s + 1, sync=False)

        def _ring(pi, carry):
            for b in range(NB):
                c = pi * NB + b
                _wait_gather(b)
                _issue_scatter(st, c, b)
                _wait_scatter(b)
                _issue_gather(st, c + NB, b)
            return carry

        lax.fori_loop(0, CPS // NB - 1, _ring, 0)
        if s + 1 < NSEG:
            _wait_idx(s + 1)
        for b in range(NB):
            _wait_gather(b)
            _issue_scatter(st, CPS - NB + b, b)
            _wait_scatter(b)
            if s + 1 < NSEG:
                _issue_gather((s + 1) % 2, b, b)
    plsc.subcore_barrier()

    for k in range(RPT // CB):
        r0 = base + k * CB
        pltpu.sync_copy(acnt.at[pl.ds(r0, CB)], cbuf)

        @pl.when(cid == 0)
        def _():
            pltpu.async_copy(cbuf, cout0.at[pl.ds(r0, CB)], ssem0)

        @pl.when(cid == 1)
        def _():
            pltpu.async_copy(cbuf, cout1.at[pl.ds(r0, CB)], ssem0)

        pltpu.make_async_copy(cbuf, cout0.at[pl.ds(r0, CB)], ssem0).wait()
    for k in range(NZF + 1):
        b = k % NB
        n = C if k < NZF else ZT
        r0 = base + k * C
        if k >= NB:
            pltpu.make_async_copy(rows[b], out0.at[pl.ds(0, C)],
                                  gsem[b]).wait()
        bounce = rows[b] if n == C else rows[b].at[pl.ds(0, ZT)]
        pltpu.sync_copy(asum.at[pl.ds(r0, n)], bounce)

        @pl.when(cid == 0)
        def _():
            pltpu.async_copy(bounce, out0.at[pl.ds(r0, n)], gsem[b])

        @pl.when(cid == 1)
        def _():
            pltpu.async_copy(bounce, out1.at[pl.ds(r0, n)], gsem[b])

    for k in range(NB):
        n = C if (NZF + 1 - NB + k) < NZF else ZT
        pltpu.make_async_copy(rows[0].at[pl.ds(0, n)],
                              out0.at[pl.ds(0, n)],
                              gsem[(NZF + 1 - NB + k) % NB]).wait()


_sc_seg_sum = functools.partial(
    pl.kernel,
    out_type=(jax.ShapeDtypeStruct((N_CONTENT, D), jnp.bfloat16),
              jax.ShapeDtypeStruct((N_CONTENT, D), jnp.bfloat16),
              jax.ShapeDtypeStruct((N_CONTENT, CW), jnp.float32),
              jax.ShapeDtypeStruct((N_CONTENT, CW), jnp.float32)),
    mesh=plsc.VectorSubcoreMesh(core_axis_name="c", subcore_axis_name="s"),
    scratch_types=[
        pltpu.VMEM((CPS, C), jnp.int32),
        pltpu.VMEM((CPS, C), jnp.int32),
        pltpu.VMEM((CPS, C), jnp.int32),
        pltpu.VMEM((CPS, C), jnp.int32),
    ] + [pltpu.VMEM((C, D), jnp.bfloat16) for _ in range(NB)] + [
        pltpu.VMEM((C, CW), jnp.float32),
        pltpu.VMEM((CB, CW), jnp.float32),
        pltpu.VMEM_SHARED((N_CONTENT, D), jnp.bfloat16),
        pltpu.VMEM_SHARED((N_CONTENT, CW), jnp.float32),
    ] + [pltpu.SemaphoreType.DMA for _ in range(2 * NB + 2)],
    compiler_params=pltpu.CompilerParams(use_tc_tiling_on_sc=False),
)(_sc_body)


_RB = 2000


def _final_body(p0_ref, p1_ref, c0_ref, c1_ref, xc_ref, wc_ref, bc_ref,
                wl_ref, bl_ref, wr_ref, wo_ref, bo_ref, o_ref):
    hc = jnp.maximum(_bdot(xc_ref[...], wc_ref[...]) + bc_ref[...], 0.0)
    wo = wo_ref[...]
    wlo = _bdot(wl_ref[...], wo)
    wro = _bdot(wr_ref[...], wo)
    bb = _bdot(bl_ref[...], wo) + bo_ref[...]
    s = (p0_ref[...].astype(jnp.float32) + p1_ref[...].astype(jnp.float32))
    cnt = c0_ref[:, 0:1] + c1_ref[:, 0:1]
    mean = s / jnp.maximum(cnt, 1.0)
    o_ref[...] = _bdot(mean, wlo) + _bdot(hc, wro) + bb


def _final(p0, p1, c0, c1, x_content, W_content, b_content, W_l, b_l, W_r,
           W_out, b_out):
    full = lambda i: (0, 0)
    return pl.pallas_call(
        _final_body,
        grid=(N_CONTENT // _RB,),
        in_specs=[
            pl.BlockSpec((_RB, D), lambda i: (i, 0)),
            pl.BlockSpec((_RB, D), lambda i: (i, 0)),
            pl.BlockSpec((_RB, CW), lambda i: (i, 0)),
            pl.BlockSpec((_RB, CW), lambda i: (i, 0)),
            pl.BlockSpec((_RB, D), lambda i: (i, 0)),
            pl.BlockSpec((D, D), full),
            pl.BlockSpec((1, D), full),
            pl.BlockSpec((D, D), full),
            pl.BlockSpec((1, D), full),
            pl.BlockSpec((D, D), full),
            pl.BlockSpec((D, D), full),
            pl.BlockSpec((1, D), full),
        ],
        out_specs=pl.BlockSpec((_RB, D), lambda i: (i, 0)),
        out_shape=jax.ShapeDtypeStruct((N_CONTENT, D), jnp.float32),
    )(p0, p1, c0, c1, x_content, W_content, b_content.reshape(1, D),
      W_l, b_l.reshape(1, D), W_r, W_out, b_out.reshape(1, D))


def kernel(x_content, x_user, edge_index, W_content, b_content, W_user, b_user,
           W_l, b_l, W_r, W_out, b_out):
    hu = _lin_user(x_user, W_user, b_user)
    eidx = edge_index.reshape(2, NW, NCHUNK, C)
    p0, p1, c0, c1 = _sc_seg_sum(hu, eidx)
    return _final(p0, p1, c0, c1, x_content, W_content, b_content, W_l, b_l,
                  W_r, W_out, b_out)
